# baseline (speedup 1.0000x reference)
import functools

import jax
import jax.numpy as jnp
from jax import lax
from jax.experimental import pallas as pl
from jax.experimental.pallas import tpu as pltpu
from jax.experimental.pallas import tpu_sc as plsc

N_NODES = 10000
D = 1024
NCHUNK = 8
CW = 128
NS = 16
NCORE = 2
B = 4096

_BN_SCALE = float(1.0 / (1.0 + 1e-5) ** 0.5)
_RRELU_SLOPE = (1.0 / 8.0 + 1.0 / 3.0) / 2.0


def _leaky(x):
    return jnp.where(x >= 0, x, 0.01 * x)


def _mesh():
    return plsc.VectorSubcoreMesh(core_axis_name="c", subcore_axis_name="s")


NPAD = 10240


def _deg_reduce_write(spm, sid, core_row, deg_hbm, red_v, tmp_v):
    base = sid * 640
    def zrow(t, _):
        red_v[pl.ds(t * 16, 16)] = jnp.full((16,), 2.0, jnp.float32)
        return 0
    lax.fori_loop(0, 40, zrow, 0)
    def srow(j, _):
        pltpu.sync_copy(spm.at[j, pl.ds(base, 640)], tmp_v)
        def add(t, _):
            red_v[pl.ds(t * 16, 16)] = (red_v[pl.ds(t * 16, 16)]
                                        + tmp_v[pl.ds(t * 16, 16)])
            return 0
        lax.fori_loop(0, 40, add, 0)
        return 0
    lax.fori_loop(0, NS, srow, 0)
    pltpu.sync_copy(red_v, deg_hbm.at[core_row, pl.ds(base, 640)])


def _sc_deg(dst_d3, w_d3, dst_p3, w_p3, g_d, g_p):
    def body(dst_d, w_d, dst_p, w_p, deg_hbm,
             dstv_d, wv_d, dstv_p, wv_p, deg_v, red_v, tmp_v, spm):
        cid = lax.axis_index("c")
        sid = lax.axis_index("s")
        def z(i, _):
            deg_v[pl.ds(i * 16, 16)] = jnp.zeros((16,), jnp.float32)
            return 0
        lax.fori_loop(0, NPAD // 16, z, 0)

        lanes = lax.iota(jnp.int32, 16)

        def accumulate(dstv, wv, n_grp):
            def grp(g, _):
                def b16(b, _):
                    dst16 = dstv[g, pl.ds(b * 16, 16)]
                    w16 = wv[g, pl.ds(b * 16, 16)]
                    for t in range(16):
                        d = dst16[t]
                        base = d & ~15
                        off = d & 15
                        sl = pl.ds(base, 16)
                        deg_v[sl] = deg_v[sl] + jnp.where(
                            lanes == off, w16[t], 0.0)
                    return 0
                return lax.fori_loop(0, 8, b16, 0)
            lax.fori_loop(0, n_grp, grp, 0)

        @pl.when(cid == 0)
        def _():
            pltpu.sync_copy(dst_d.at[sid], dstv_d)
            pltpu.sync_copy(w_d.at[sid], wv_d)
            accumulate(dstv_d, wv_d, g_d)

        @pl.when(cid == 1)
        def _():
            pltpu.sync_copy(dst_p.at[sid], dstv_p)
            pltpu.sync_copy(w_p.at[sid], wv_p)
            accumulate(dstv_p, wv_p, g_p)

        pltpu.sync_copy(deg_v, spm.at[sid])
        plsc.subcore_barrier()
        _deg_reduce_write(spm, sid, cid, deg_hbm, red_v, tmp_v)

    return pl.kernel(
        body,
        out_type=jax.ShapeDtypeStruct((NCORE, NPAD), jnp.float32),
        mesh=_mesh(),
        compiler_params=pltpu.CompilerParams(needs_layout_passes=False),
        scratch_types=[
            pltpu.VMEM((g_d, 128), jnp.int32),
            pltpu.VMEM((g_d, 128), jnp.float32),
            pltpu.VMEM((g_p, 128), jnp.int32),
            pltpu.VMEM((g_p, 128), jnp.float32),
            pltpu.VMEM((NPAD,), jnp.float32),
            pltpu.VMEM((640,), jnp.float32),
            pltpu.VMEM((640,), jnp.float32),
            pltpu.VMEM_SHARED((NS, NPAD), jnp.float32),
        ],
    )(dst_d3, w_d3, dst_p3, w_p3)


HALF = N_NODES // 2


def _sc_msg(hs_flat, src3, dst3, w3, n_grp):
    cap = n_grp * 128

    def body(hs, src_h, dst_h, w_h, acc_hbm,
             w_v, srcc, dstc1, wc, dstc2, gidx2,
             rows0, rows1, rows2, zero_v, acc_sp,
             gs0, gs1, gs2, ss0, ss1, ss2):
        rows = (rows0, rows1, rows2)
        gsem = (gs0, gs1, gs2)
        ssem = (ss0, ss1, ss2)
        cid = lax.axis_index("c")
        sid = lax.axis_index("s")
        lo = cid * HALF
        pltpu.sync_copy(src_h.at[sid], gidx2)
        pltpu.sync_copy(dst_h.at[sid], dstc2)
        pltpu.sync_copy(w_h.at[sid], w_v)

        def zc(i, _):
            sl = pl.ds(i * 16, 16)
            srcc[sl] = jnp.zeros((16,), jnp.int32)
            dstc1[sl] = jnp.zeros((16,), jnp.int32)
            wc[sl] = jnp.zeros((16,), jnp.float32)
            return 0
        lax.fori_loop(0, cap // 16 + 1, zc, 0)
        def zr(r, _):
            for t in range(8):
                zero_v[r, pl.ds(t * 16, 16)] = jnp.zeros((16,), jnp.float32)
            return 0
        lax.fori_loop(0, 80, zr, 0)

        def comp_g(g, off):
            def comp_b(b, off):
                sl = pl.ds(b * 16, 16)
                d16 = dstc2[g, sl]
                s16 = gidx2[g, sl]
                w16 = w_v[g, sl]
                m = (d16 >= lo) & (d16 < lo + HALF)
                osl = pl.ds(off, 16)
                plsc.store_compressed(srcc.at[osl], s16, mask=m)
                plsc.store_compressed(dstc1.at[osl], d16 - lo, mask=m)
                plsc.store_compressed(wc.at[osl], w16, mask=m)
                return off + plsc.all_reduce_population_count(m)[0]
            return lax.fori_loop(0, 8, comp_b, off)
        m_edges = lax.fori_loop(0, n_grp, comp_g, jnp.int32(0))
        m_grp = (m_edges + 127) // 128

        def cp_g(g, _):
            for t in range(8):
                dstc2[g, pl.ds(t * 16, 16)] = dstc1[pl.ds(g * 128 + t * 16, 16)]
            return 0
        lax.fori_loop(0, m_grp, cp_g, 0)

        def scale(rv, g):
            def b16(b, _):
                w16 = wc[pl.ds(g * 128 + b * 16, 16)]
                for t in range(16):
                    s = w16[t]
                    e = b * 16 + t
                    for t2 in range(8):
                        sl = pl.ds(t2 * 16, 16)
                        rv[e, sl] = rv[e, sl] * s
                return 0
            lax.fori_loop(0, 8, b16, 0)

        base = sid * 312

        def chunk(k, _):
            hoff = k * N_NODES
            @pl.when(sid < NS - 1)
            def _():
                for piece in range(3):
                    pltpu.sync_copy(zero_v,
                                    acc_sp.at[pl.ds(base + piece * 80, 80)])
                pltpu.sync_copy(zero_v.at[pl.ds(0, 72)],
                                acc_sp.at[pl.ds(base + 240, 72)])

            @pl.when(sid == NS - 1)
            def _():
                for piece in range(4):
                    pltpu.sync_copy(zero_v,
                                    acc_sp.at[pl.ds(base + piece * 80, 80)])

            def gi(g, _):
                for t in range(8):
                    gidx2[g, pl.ds(t * 16, 16)] = (
                        srcc[pl.ds(g * 128 + t * 16, 16)] + hoff)
                return 0
            lax.fori_loop(0, m_grp, gi, 0)
            plsc.subcore_barrier()

            def grp_s(g, _):
                pltpu.async_copy(hs.at[gidx2.at[g]], rows[0], gsem[0]).wait()
                scale(rows[0], g)
                pltpu.sync_copy(rows[0], acc_sp.at[dstc2.at[g]], add=True)
                return 0
            lax.fori_loop(0, m_grp, grp_s, 0)
            plsc.subcore_barrier()

            wb = hoff + cid * HALF + base

            @pl.when(sid < NS - 1)
            def _():
                pltpu.sync_copy(acc_sp.at[pl.ds(base, 312)],
                                acc_hbm.at[pl.ds(wb, 312)])

            @pl.when(sid == NS - 1)
            def _():
                pltpu.sync_copy(acc_sp.at[pl.ds(base, 320)],
                                acc_hbm.at[pl.ds(wb, 320)])

            plsc.subcore_barrier()
            return 0
        lax.fori_loop(0, NCHUNK, chunk, 0)

    return pl.kernel(
        body,
        out_type=jax.ShapeDtypeStruct((NCHUNK * N_NODES, CW), jnp.float32),
        mesh=_mesh(),
        compiler_params=pltpu.CompilerParams(needs_layout_passes=False),
        scratch_types=[
            pltpu.VMEM((n_grp, 128), jnp.float32),
            pltpu.VMEM((cap + 16,), jnp.int32),
            pltpu.VMEM((cap + 16,), jnp.int32),
            pltpu.VMEM((cap + 16,), jnp.float32),
            pltpu.VMEM((n_grp, 128), jnp.int32),
            pltpu.VMEM((n_grp, 128), jnp.int32),
            pltpu.VMEM((128, CW), jnp.float32),
            pltpu.VMEM((128, CW), jnp.float32),
            pltpu.VMEM((128, CW), jnp.float32),
            pltpu.VMEM((80, CW), jnp.float32),
            pltpu.VMEM_SHARED((HALF, CW), jnp.float32),
        ] + [pltpu.SemaphoreType.DMA] * 6,
    )(hs_flat, src3, dst3, w3)


def _sc_gather(hs_d, acc_d, hs_p, acc_p, dinv2, didx2, pidx2, bd2, bp2):
    def per_core(hs, acc, idx_h, b_h, out_hbm, dinv_row,
                 dinv_v, idx_v, gidx_v, dr_v, b_v, rows_h, rows_a, out_v,
                 sem, sid):
        pltpu.sync_copy(dinv_row, dinv_v)
        pltpu.sync_copy(idx_h.at[sid], idx_v)
        pltpu.sync_copy(b_h, b_v)
        for g in range(2):
            for t in range(8):
                sl = pl.ds(t * 16, 16)
                dr_v[g, sl] = plsc.load_gather(dinv_v, [idx_v[g, sl]])

        def chunk(k, _):
            off = k * N_NODES
            for g in range(2):
                for t in range(8):
                    sl = pl.ds(t * 16, 16)
                    gidx_v[g, sl] = idx_v[g, sl] + off
            for g in range(2):
                cp1 = pltpu.async_copy(hs.at[gidx_v.at[g]], rows_h, sem)
                cp1.wait()
                cp2 = pltpu.async_copy(acc.at[gidx_v.at[g]], rows_a, sem)
                cp2.wait()
                def b16(b, _):
                    dr16 = dr_v[g, pl.ds(b * 16, 16)]
                    for t in range(16):
                        s = dr16[t]
                        e = b * 16 + t
                        for t2 in range(8):
                            sl = pl.ds(t2 * 16, 16)
                            x = (2.0 * rows_h[e, sl] + rows_a[e, sl]) * s
                            x = x + b_v[k, sl]
                            out_v[e, sl] = jnp.where(x >= 0, x, 0.01 * x)
                    return 0
                lax.fori_loop(0, 8, b16, 0)
                pltpu.sync_copy(
                    out_v,
                    out_hbm.at[pl.ds(sid * 256 + g * 128, 128),
                               pl.ds(k * CW, CW)])
            return 0
        lax.fori_loop(0, NCHUNK, chunk, 0)

    def body(hs_d_r, acc_d_r, hs_p_r, acc_p_r, dinv_r, didx_r, pidx_r,
             bd_r, bp_r, ec_hbm, go_hbm,
             dinv_v, idx_v, gidx_v, dr_v, b_v, rows_h, rows_a, out_v, sem):
        cid = lax.axis_index("c")
        sid = lax.axis_index("s")

        @pl.when(cid == 0)
        def _():
            per_core(hs_d_r, acc_d_r, didx_r, bd_r, ec_hbm, dinv_r.at[0],
                     dinv_v, idx_v, gidx_v, dr_v, b_v, rows_h, rows_a,
                     out_v, sem, sid)

        @pl.when(cid == 1)
        def _():
            per_core(hs_p_r, acc_p_r, pidx_r, bp_r, go_hbm, dinv_r.at[1],
                     dinv_v, idx_v, gidx_v, dr_v, b_v, rows_h, rows_a,
                     out_v, sem, sid)

    return pl.kernel(
        body,
        out_type=[jax.ShapeDtypeStruct((B, D), jnp.float32),
                  jax.ShapeDtypeStruct((B, D), jnp.float32)],
        mesh=_mesh(),
        compiler_params=pltpu.CompilerParams(needs_layout_passes=False),
        scratch_types=[
            pltpu.VMEM((N_NODES,), jnp.float32),
            pltpu.VMEM((2, 128), jnp.int32),
            pltpu.VMEM((2, 128), jnp.int32),
            pltpu.VMEM((2, 128), jnp.float32),
            pltpu.VMEM((NCHUNK, CW), jnp.float32),
            pltpu.VMEM((128, CW), jnp.float32),
            pltpu.VMEM((128, CW), jnp.float32),
            pltpu.VMEM((128, CW), jnp.float32),
            pltpu.SemaphoreType.DMA,
        ],
    )(hs_d, acc_d, hs_p, acc_p, dinv2, didx2, pidx2, bd2, bp2)


def _dinv_body(deg_ref, o_ref):
    d = deg_ref[...]
    o_ref[...] = jnp.where(d > 0, lax.rsqrt(d), 0.0)


def _tc_dinv(deg2):
    return pl.pallas_call(
        _dinv_body,
        out_shape=jax.ShapeDtypeStruct((NCORE, NPAD), jnp.float32),
    )(deg2)


def _mmc_body(x_ref, w_ref, dinv_ref, o_ref):
    x = x_ref[...]
    dv = dinv_ref[...]
    for j in range(NCHUNK):
        o_ref[j] = dv * jnp.dot(x, w_ref[:, j * CW:(j + 1) * CW],
                                preferred_element_type=jnp.float32)


def _tc_matmul_chunked(x, w, dinv_col, bm):
    m, k = x.shape
    grid = (m // bm,)
    return pl.pallas_call(
        _mmc_body,
        grid=grid,
        in_specs=[
            pl.BlockSpec((bm, k), lambda i: (i, 0)),
            pl.BlockSpec((k, D), lambda i: (0, 0)),
            pl.BlockSpec((bm, 1), lambda i: (i, 0)),
        ],
        out_specs=pl.BlockSpec((NCHUNK, bm, CW), lambda i: (0, i, 0)),
        out_shape=jax.ShapeDtypeStruct((NCHUNK, m, CW), jnp.float32),
    )(x, w, dinv_col)


def _mlp_body(dv_ref, pe_ref, ec_ref, go_ref,
              w1a_ref, w1b_ref, w1c_ref, w1d_ref, b1_ref, g1_ref, be1_ref,
              w2_ref, b2_ref, g2_ref, be2_ref,
              w3_ref, b3_ref, g3_ref, be3_ref,
              w4_ref, b4_ref,
              y_ref, feat_ref):
    h = jnp.dot(dv_ref[...], w1a_ref[...], preferred_element_type=jnp.float32)
    h += jnp.dot(pe_ref[...], w1b_ref[...], preferred_element_type=jnp.float32)
    h += jnp.dot(ec_ref[...], w1c_ref[...], preferred_element_type=jnp.float32)
    h += jnp.dot(go_ref[...], w1d_ref[...], preferred_element_type=jnp.float32)
    h = h + b1_ref[...]
    h = _leaky(h * _BN_SCALE * g1_ref[...] + be1_ref[...])

    f = jnp.dot(h, w2_ref[...], preferred_element_type=jnp.float32) + b2_ref[...]
    f = _leaky(f * _BN_SCALE * g2_ref[...] + be2_ref[...])
    feat_ref[...] = f

    o = jnp.dot(f, w3_ref[...], preferred_element_type=jnp.float32) + b3_ref[...]
    o = jnp.where(o >= 0, o, o * _RRELU_SLOPE)
    o = o * _BN_SCALE * g3_ref[...] + be3_ref[...]

    y_ref[...] = jnp.dot(o, w4_ref[...], preferred_element_type=jnp.float32) \
        + b4_ref[...]


def _pallas_mlp(dv, pe, ec, go, W1, b1, g1, be1, W2, b2, g2, be2,
                W3, b3, g3, be3, W4, b4):
    bm = 512
    grid = (B // bm,)
    w1a = W1[:300]
    w1b = W1[300:1324]
    w1c = W1[1324:2348]
    w1d = W1[2348:]
    row = lambda v: v.reshape(1, -1)

    def full(a):
        return pl.BlockSpec(a.shape, lambda i: (0,) * a.ndim)

    args = (dv, pe, ec, go, w1a, w1b, w1c, w1d, row(b1), row(g1), row(be1),
            W2, row(b2), row(g2), row(be2), W3, row(b3), row(g3), row(be3),
            W4, row(b4))
    in_specs = [
        pl.BlockSpec((bm, 300), lambda i: (i, 0)),
        pl.BlockSpec((bm, 1024), lambda i: (i, 0)),
        pl.BlockSpec((bm, 1024), lambda i: (i, 0)),
        pl.BlockSpec((bm, 1024), lambda i: (i, 0)),
    ] + [full(a) for a in args[4:]]
    return pl.pallas_call(
        _mlp_body,
        grid=grid,
        in_specs=in_specs,
        out_specs=[
            pl.BlockSpec((bm, 1), lambda i: (i, 0)),
            pl.BlockSpec((bm, 512), lambda i: (i, 0)),
        ],
        out_shape=[
            jax.ShapeDtypeStruct((B, 1), jnp.float32),
            jax.ShapeDtypeStruct((B, 512), jnp.float32),
        ],
    )(*args)


def _pad_edges(edge_index, edge_weight, n_grp):
    e = edge_index.shape[1]
    cap = NS * n_grp * 128
    src = jnp.zeros((cap,), jnp.int32).at[:e].set(
        edge_index[0].astype(jnp.int32))
    dst = jnp.full((cap,), 10016, jnp.int32).at[:e].set(
        edge_index[1].astype(jnp.int32))
    w = jnp.zeros((cap,), jnp.float32).at[:e].set(edge_weight)
    return (src.reshape(NS, n_grp, 128), dst.reshape(NS, n_grp, 128),
            w.reshape(NS, n_grp, 128))


def kernel(d_index, p_index, d_vecs, p_embeddings, y, d_ecfps, d_edge_index,
           d_edge_weight, p_gos, p_edge_index, p_edge_weight, Wd, bd, Wp, bp,
           W1, b1, g1, be1, W2, b2, g2, be2, W3, b3, g3, be3, W4, b4):
    g_d = -(-d_edge_index.shape[1] // (NS * 128))
    g_p = -(-p_edge_index.shape[1] // (NS * 128))
    src_d3, dst_d3, w_d3 = _pad_edges(d_edge_index, d_edge_weight, g_d)
    src_p3, dst_p3, w_p3 = _pad_edges(p_edge_index, p_edge_weight, g_p)

    deg2 = _sc_deg(dst_d3, w_d3, dst_p3, w_p3, g_d, g_p)
    dinv2 = _tc_dinv(deg2)[:, :N_NODES]

    hs_d = _tc_matmul_chunked(d_ecfps, Wd, dinv2[0].reshape(-1, 1), 1000)
    hs_p = _tc_matmul_chunked(p_gos, Wp, dinv2[1].reshape(-1, 1), 1000)
    hs_d_flat = hs_d.reshape(NCHUNK * N_NODES, CW)
    hs_p_flat = hs_p.reshape(NCHUNK * N_NODES, CW)

    acc_d = _sc_msg(hs_d_flat, src_d3, dst_d3, w_d3, g_d)
    acc_p = _sc_msg(hs_p_flat, src_p3, dst_p3, w_p3, g_p)

    didx2 = d_index.astype(jnp.int32).reshape(NS, 2, 128)
    pidx2 = p_index.astype(jnp.int32).reshape(NS, 2, 128)
    ec, go = _sc_gather(hs_d_flat, acc_d, hs_p_flat, acc_p, dinv2,
                        didx2, pidx2,
                        bd.reshape(NCHUNK, CW), bp.reshape(NCHUNK, CW))

    y_out, feature = _pallas_mlp(d_vecs, p_embeddings, ec, go,
                                 W1, b1, g1, be1, W2, b2, g2, be2,
                                 W3, b3, g3, be3, W4, b4)
    return (y_out, feature)


# speedup vs baseline: 1.6600x; 1.6600x over previous
import functools

import jax
import jax.numpy as jnp
from jax import lax
from jax.experimental import pallas as pl
from jax.experimental.pallas import tpu as pltpu
from jax.experimental.pallas import tpu_sc as plsc

N_NODES = 10000
D = 1024
NCHUNK = 8
CW = 128
NS = 16
NCORE = 2
B = 4096

_BN_SCALE = float(1.0 / (1.0 + 1e-5) ** 0.5)
_RRELU_SLOPE = (1.0 / 8.0 + 1.0 / 3.0) / 2.0


def _leaky(x):
    return jnp.where(x >= 0, x, 0.01 * x)


def _mesh():
    return plsc.VectorSubcoreMesh(core_axis_name="c", subcore_axis_name="s")


NPAD = 10240


def _deg_reduce_write(spm, sid, core_row, deg_hbm, red_v, tmp_v):
    base = sid * 640
    def zrow(t, _):
        red_v[pl.ds(t * 16, 16)] = jnp.full((16,), 2.0, jnp.float32)
        return 0
    lax.fori_loop(0, 40, zrow, 0)
    def srow(j, _):
        pltpu.sync_copy(spm.at[j, pl.ds(base, 640)], tmp_v)
        def add(t, _):
            red_v[pl.ds(t * 16, 16)] = (red_v[pl.ds(t * 16, 16)]
                                        + tmp_v[pl.ds(t * 16, 16)])
            return 0
        lax.fori_loop(0, 40, add, 0)
        return 0
    lax.fori_loop(0, NS, srow, 0)
    pltpu.sync_copy(red_v, deg_hbm.at[core_row, pl.ds(base, 640)])


def _sc_deg(dst_d3, w_d3, dst_p3, w_p3, g_d, g_p):
    def body(dst_d, w_d, dst_p, w_p, deg_hbm,
             dstv_d, wv_d, dstv_p, wv_p, deg_v, red_v, tmp_v, spm):
        cid = lax.axis_index("c")
        sid = lax.axis_index("s")
        def z(i, _):
            deg_v[pl.ds(i * 16, 16)] = jnp.zeros((16,), jnp.float32)
            return 0
        lax.fori_loop(0, NPAD // 16, z, 0)

        lanes = lax.iota(jnp.int32, 16)

        def accumulate(dstv, wv, n_grp):
            def grp(g, _):
                def b16(b, _):
                    dst16 = dstv[g, pl.ds(b * 16, 16)]
                    w16 = wv[g, pl.ds(b * 16, 16)]
                    for t in range(16):
                        d = dst16[t]
                        base = d & ~15
                        off = d & 15
                        sl = pl.ds(base, 16)
                        deg_v[sl] = deg_v[sl] + jnp.where(
                            lanes == off, w16[t], 0.0)
                    return 0
                return lax.fori_loop(0, 8, b16, 0)
            lax.fori_loop(0, n_grp, grp, 0)

        @pl.when(cid == 0)
        def _():
            pltpu.sync_copy(dst_d.at[sid], dstv_d)
            pltpu.sync_copy(w_d.at[sid], wv_d)
            accumulate(dstv_d, wv_d, g_d)

        @pl.when(cid == 1)
        def _():
            pltpu.sync_copy(dst_p.at[sid], dstv_p)
            pltpu.sync_copy(w_p.at[sid], wv_p)
            accumulate(dstv_p, wv_p, g_p)

        pltpu.sync_copy(deg_v, spm.at[sid])
        plsc.subcore_barrier()
        _deg_reduce_write(spm, sid, cid, deg_hbm, red_v, tmp_v)

    return pl.kernel(
        body,
        out_type=jax.ShapeDtypeStruct((NCORE, NPAD), jnp.float32),
        mesh=_mesh(),
        compiler_params=pltpu.CompilerParams(needs_layout_passes=False),
        scratch_types=[
            pltpu.VMEM((g_d, 128), jnp.int32),
            pltpu.VMEM((g_d, 128), jnp.float32),
            pltpu.VMEM((g_p, 128), jnp.int32),
            pltpu.VMEM((g_p, 128), jnp.float32),
            pltpu.VMEM((NPAD,), jnp.float32),
            pltpu.VMEM((640,), jnp.float32),
            pltpu.VMEM((640,), jnp.float32),
            pltpu.VMEM_SHARED((NS, NPAD), jnp.float32),
        ],
    )(dst_d3, w_d3, dst_p3, w_p3)


def _sc_msg(hs_flat, src3, dst3, w3, zeros_h, n_grp):
    def body(hs, src_h, dst_h, w_h, zeros_hbm, acc_hbm,
             dst_v, w_v, gidx2, rows0, rows1, acc_sp,
             gs0, gs1, ss0, ss1):
        rows = (rows0, rows1)
        gsem = (gs0, gs1)
        ssem = (ss0, ss1)
        cid = lax.axis_index("c")
        sid = lax.axis_index("s")
        pltpu.sync_copy(dst_h.at[sid], dst_v)
        pltpu.sync_copy(w_h.at[sid], w_v)

        def scale(rv, g):
            def b16(b, _):
                w16 = w_v[g, pl.ds(b * 16, 16)]
                for t in range(16):
                    s = w16[t]
                    e = b * 16 + t
                    for t2 in range(8):
                        sl = pl.ds(t2 * 16, 16)
                        rv[e, sl] = rv[e, sl] * s
                return 0
            lax.fori_loop(0, 8, b16, 0)

        base = sid * 624

        def chunk(j, _):
            k = 2 * j + cid
            hoff = k * N_NODES

            @pl.when(sid < NS - 1)
            def _():
                pltpu.sync_copy(zeros_hbm.at[pl.ds(0, 624)],
                                acc_sp.at[pl.ds(base, 624)])

            @pl.when(sid == NS - 1)
            def _():
                pltpu.sync_copy(zeros_hbm, acc_sp.at[pl.ds(base, 640)])

            pltpu.sync_copy(src_h.at[sid], gidx2)
            def gi(g, _):
                for t in range(8):
                    sl = pl.ds(t * 16, 16)
                    gidx2[g, sl] = gidx2[g, sl] + hoff
                return 0
            lax.fori_loop(0, n_grp, gi, 0)
            plsc.subcore_barrier()

            pltpu.async_copy(hs.at[gidx2.at[0]], rows[0], gsem[0])

            def round2(r, _):
                for b in range(2):
                    g = 2 * r + b
                    nb = 1 - b
                    pltpu.make_async_copy(hs.at[gidx2.at[g]], rows[b],
                                          gsem[b]).wait()
                    scale(rows[b], g)
                    pltpu.async_copy(rows[b], acc_sp.at[dst_v.at[g]],
                                     ssem[b], add=True)

                    @pl.when(g + 1 < n_grp)
                    def _():
                        @pl.when(g >= 1)
                        def _():
                            pltpu.make_async_copy(rows[nb],
                                                  acc_sp.at[dst_v.at[g - 1]],
                                                  ssem[nb]).wait()
                        pltpu.async_copy(hs.at[gidx2.at[g + 1]], rows[nb],
                                         gsem[nb])
                return 0
            lax.fori_loop(0, n_grp // 2, round2, 0)
            pltpu.make_async_copy(rows[0], acc_sp.at[dst_v.at[n_grp - 2]],
                                  ssem[0]).wait()
            pltpu.make_async_copy(rows[1], acc_sp.at[dst_v.at[n_grp - 1]],
                                  ssem[1]).wait()
            plsc.subcore_barrier()

            @pl.when(sid < NS - 1)
            def _():
                pltpu.sync_copy(acc_sp.at[pl.ds(base, 624)],
                                acc_hbm.at[pl.ds(hoff + base, 624)])

            @pl.when(sid == NS - 1)
            def _():
                pltpu.sync_copy(acc_sp.at[pl.ds(base, 640)],
                                acc_hbm.at[pl.ds(hoff + base, 640)])

            plsc.subcore_barrier()
            return 0
        lax.fori_loop(0, NCHUNK // NCORE, chunk, 0)

    return pl.kernel(
        body,
        out_type=jax.ShapeDtypeStruct((NCHUNK * N_NODES, CW), jnp.float32),
        mesh=_mesh(),
        compiler_params=pltpu.CompilerParams(needs_layout_passes=False),
        scratch_types=[
            pltpu.VMEM((n_grp, 128), jnp.int32),
            pltpu.VMEM((n_grp, 128), jnp.float32),
            pltpu.VMEM((n_grp, 128), jnp.int32),
            pltpu.VMEM((128, CW), jnp.float32),
            pltpu.VMEM((128, CW), jnp.float32),
            pltpu.VMEM_SHARED((NPAD, CW), jnp.float32),
        ] + [pltpu.SemaphoreType.DMA] * 4,
    )(hs_flat, src3, dst3, w3, zeros_h)


def _sc_gather(hs_d, acc_d, hs_p, acc_p, dinv2, didx2, pidx2, bd2, bp2):
    def per_core(hs, acc, idx_h, b_h, out_hbm, dinv_row,
                 dinv_v, idx_v, gidx_v, dr_v, b_v, rows_h, rows_a, out_v,
                 sem, sid):
        pltpu.sync_copy(dinv_row, dinv_v)
        pltpu.sync_copy(idx_h.at[sid], idx_v)
        pltpu.sync_copy(b_h, b_v)
        for g in range(2):
            for t in range(8):
                sl = pl.ds(t * 16, 16)
                dr_v[g, sl] = plsc.load_gather(dinv_v, [idx_v[g, sl]])

        def chunk(k, _):
            off = k * N_NODES
            for g in range(2):
                for t in range(8):
                    sl = pl.ds(t * 16, 16)
                    gidx_v[g, sl] = idx_v[g, sl] + off
            for g in range(2):
                cp1 = pltpu.async_copy(hs.at[gidx_v.at[g]], rows_h, sem)
                cp1.wait()
                cp2 = pltpu.async_copy(acc.at[gidx_v.at[g]], rows_a, sem)
                cp2.wait()
                def b16(b, _):
                    dr16 = dr_v[g, pl.ds(b * 16, 16)]
                    for t in range(16):
                        s = dr16[t]
                        e = b * 16 + t
                        for t2 in range(8):
                            sl = pl.ds(t2 * 16, 16)
                            x = (2.0 * rows_h[e, sl] + rows_a[e, sl]) * s
                            x = x + b_v[k, sl]
                            out_v[e, sl] = jnp.where(x >= 0, x, 0.01 * x)
                    return 0
                lax.fori_loop(0, 8, b16, 0)
                pltpu.sync_copy(
                    out_v,
                    out_hbm.at[pl.ds(sid * 256 + g * 128, 128),
                               pl.ds(k * CW, CW)])
            return 0
        lax.fori_loop(0, NCHUNK, chunk, 0)

    def body(hs_d_r, acc_d_r, hs_p_r, acc_p_r, dinv_r, didx_r, pidx_r,
             bd_r, bp_r, ec_hbm, go_hbm,
             dinv_v, idx_v, gidx_v, dr_v, b_v, rows_h, rows_a, out_v, sem):
        cid = lax.axis_index("c")
        sid = lax.axis_index("s")

        @pl.when(cid == 0)
        def _():
            per_core(hs_d_r, acc_d_r, didx_r, bd_r, ec_hbm, dinv_r.at[0],
                     dinv_v, idx_v, gidx_v, dr_v, b_v, rows_h, rows_a,
                     out_v, sem, sid)

        @pl.when(cid == 1)
        def _():
            per_core(hs_p_r, acc_p_r, pidx_r, bp_r, go_hbm, dinv_r.at[1],
                     dinv_v, idx_v, gidx_v, dr_v, b_v, rows_h, rows_a,
                     out_v, sem, sid)

    return pl.kernel(
        body,
        out_type=[jax.ShapeDtypeStruct((B, D), jnp.float32),
                  jax.ShapeDtypeStruct((B, D), jnp.float32)],
        mesh=_mesh(),
        compiler_params=pltpu.CompilerParams(needs_layout_passes=False),
        scratch_types=[
            pltpu.VMEM((N_NODES,), jnp.float32),
            pltpu.VMEM((2, 128), jnp.int32),
            pltpu.VMEM((2, 128), jnp.int32),
            pltpu.VMEM((2, 128), jnp.float32),
            pltpu.VMEM((NCHUNK, CW), jnp.float32),
            pltpu.VMEM((128, CW), jnp.float32),
            pltpu.VMEM((128, CW), jnp.float32),
            pltpu.VMEM((128, CW), jnp.float32),
            pltpu.SemaphoreType.DMA,
        ],
    )(hs_d, acc_d, hs_p, acc_p, dinv2, didx2, pidx2, bd2, bp2)


def _dinv_body(deg_ref, o_ref):
    d = deg_ref[...]
    o_ref[...] = jnp.where(d > 0, lax.rsqrt(d), 0.0)


def _tc_dinv(deg2):
    return pl.pallas_call(
        _dinv_body,
        out_shape=jax.ShapeDtypeStruct((NCORE, NPAD), jnp.float32),
    )(deg2)


def _mmc_body(x_ref, w_ref, dinv_ref, o_ref):
    x = x_ref[...]
    dv = dinv_ref[...]
    for j in range(NCHUNK):
        o_ref[j] = dv * jnp.dot(x, w_ref[:, j * CW:(j + 1) * CW],
                                preferred_element_type=jnp.float32)


def _tc_matmul_chunked(x, w, dinv_col, bm):
    m, k = x.shape
    grid = (m // bm,)
    return pl.pallas_call(
        _mmc_body,
        grid=grid,
        in_specs=[
            pl.BlockSpec((bm, k), lambda i: (i, 0)),
            pl.BlockSpec((k, D), lambda i: (0, 0)),
            pl.BlockSpec((bm, 1), lambda i: (i, 0)),
        ],
        out_specs=pl.BlockSpec((NCHUNK, bm, CW), lambda i: (0, i, 0)),
        out_shape=jax.ShapeDtypeStruct((NCHUNK, m, CW), jnp.float32),
    )(x, w, dinv_col)


def _mlp_body(dv_ref, pe_ref, ec_ref, go_ref,
              w1a_ref, w1b_ref, w1c_ref, w1d_ref, b1_ref, g1_ref, be1_ref,
              w2_ref, b2_ref, g2_ref, be2_ref,
              w3_ref, b3_ref, g3_ref, be3_ref,
              w4_ref, b4_ref,
              y_ref, feat_ref):
    h = jnp.dot(dv_ref[...], w1a_ref[...], preferred_element_type=jnp.float32)
    h += jnp.dot(pe_ref[...], w1b_ref[...], preferred_element_type=jnp.float32)
    h += jnp.dot(ec_ref[...], w1c_ref[...], preferred_element_type=jnp.float32)
    h += jnp.dot(go_ref[...], w1d_ref[...], preferred_element_type=jnp.float32)
    h = h + b1_ref[...]
    h = _leaky(h * _BN_SCALE * g1_ref[...] + be1_ref[...])

    f = jnp.dot(h, w2_ref[...], preferred_element_type=jnp.float32) + b2_ref[...]
    f = _leaky(f * _BN_SCALE * g2_ref[...] + be2_ref[...])
    feat_ref[...] = f

    o = jnp.dot(f, w3_ref[...], preferred_element_type=jnp.float32) + b3_ref[...]
    o = jnp.where(o >= 0, o, o * _RRELU_SLOPE)
    o = o * _BN_SCALE * g3_ref[...] + be3_ref[...]

    y_ref[...] = jnp.dot(o, w4_ref[...], preferred_element_type=jnp.float32) \
        + b4_ref[...]


def _pallas_mlp(dv, pe, ec, go, W1, b1, g1, be1, W2, b2, g2, be2,
                W3, b3, g3, be3, W4, b4):
    bm = 512
    grid = (B // bm,)
    w1a = W1[:300]
    w1b = W1[300:1324]
    w1c = W1[1324:2348]
    w1d = W1[2348:]
    row = lambda v: v.reshape(1, -1)

    def full(a):
        return pl.BlockSpec(a.shape, lambda i: (0,) * a.ndim)

    args = (dv, pe, ec, go, w1a, w1b, w1c, w1d, row(b1), row(g1), row(be1),
            W2, row(b2), row(g2), row(be2), W3, row(b3), row(g3), row(be3),
            W4, row(b4))
    in_specs = [
        pl.BlockSpec((bm, 300), lambda i: (i, 0)),
        pl.BlockSpec((bm, 1024), lambda i: (i, 0)),
        pl.BlockSpec((bm, 1024), lambda i: (i, 0)),
        pl.BlockSpec((bm, 1024), lambda i: (i, 0)),
    ] + [full(a) for a in args[4:]]
    return pl.pallas_call(
        _mlp_body,
        grid=grid,
        in_specs=in_specs,
        out_specs=[
            pl.BlockSpec((bm, 1), lambda i: (i, 0)),
            pl.BlockSpec((bm, 512), lambda i: (i, 0)),
        ],
        out_shape=[
            jax.ShapeDtypeStruct((B, 1), jnp.float32),
            jax.ShapeDtypeStruct((B, 512), jnp.float32),
        ],
    )(*args)


def _pad_edges(edge_index, edge_weight, n_grp):
    e = edge_index.shape[1]
    cap = NS * n_grp * 128
    src = jnp.zeros((cap,), jnp.int32).at[:e].set(
        edge_index[0].astype(jnp.int32))
    dst = (10000 + jnp.arange(cap, dtype=jnp.int32) % 240).at[:e].set(
        edge_index[1].astype(jnp.int32))
    w = jnp.zeros((cap,), jnp.float32).at[:e].set(edge_weight)
    return (src.reshape(NS, n_grp, 128), dst.reshape(NS, n_grp, 128),
            w.reshape(NS, n_grp, 128))


def kernel(d_index, p_index, d_vecs, p_embeddings, y, d_ecfps, d_edge_index,
           d_edge_weight, p_gos, p_edge_index, p_edge_weight, Wd, bd, Wp, bp,
           W1, b1, g1, be1, W2, b2, g2, be2, W3, b3, g3, be3, W4, b4):
    g_d = -(-d_edge_index.shape[1] // (NS * 128))
    g_p = -(-p_edge_index.shape[1] // (NS * 128))
    src_d3, dst_d3, w_d3 = _pad_edges(d_edge_index, d_edge_weight, g_d)
    src_p3, dst_p3, w_p3 = _pad_edges(p_edge_index, p_edge_weight, g_p)

    deg2 = _sc_deg(dst_d3, w_d3, dst_p3, w_p3, g_d, g_p)
    dinv2 = _tc_dinv(deg2)[:, :N_NODES]

    hs_d = _tc_matmul_chunked(d_ecfps, Wd, dinv2[0].reshape(-1, 1), 1000)
    hs_p = _tc_matmul_chunked(p_gos, Wp, dinv2[1].reshape(-1, 1), 1000)
    hs_d_flat = hs_d.reshape(NCHUNK * N_NODES, CW)
    hs_p_flat = hs_p.reshape(NCHUNK * N_NODES, CW)

    zeros_h = jnp.zeros((640, CW), jnp.float32)
    acc_d = _sc_msg(hs_d_flat, src_d3, dst_d3, w_d3, zeros_h, g_d)
    acc_p = _sc_msg(hs_p_flat, src_p3, dst_p3, w_p3, zeros_h, g_p)

    didx2 = d_index.astype(jnp.int32).reshape(NS, 2, 128)
    pidx2 = p_index.astype(jnp.int32).reshape(NS, 2, 128)
    ec, go = _sc_gather(hs_d_flat, acc_d, hs_p_flat, acc_p, dinv2,
                        didx2, pidx2,
                        bd.reshape(NCHUNK, CW), bp.reshape(NCHUNK, CW))

    y_out, feature = _pallas_mlp(d_vecs, p_embeddings, ec, go,
                                 W1, b1, g1, be1, W2, b2, g2, be2,
                                 W3, b3, g3, be3, W4, b4)
    return (y_out, feature)


# speedup vs baseline: 1.7181x; 1.0350x over previous
import functools

import jax
import jax.numpy as jnp
from jax import lax
from jax.experimental import pallas as pl
from jax.experimental.pallas import tpu as pltpu
from jax.experimental.pallas import tpu_sc as plsc

N_NODES = 10000
D = 1024
NCHUNK = 8
CW = 128
NS = 16
NCORE = 2
B = 4096

_BN_SCALE = float(1.0 / (1.0 + 1e-5) ** 0.5)
_RRELU_SLOPE = (1.0 / 8.0 + 1.0 / 3.0) / 2.0


def _leaky(x):
    return jnp.where(x >= 0, x, 0.01 * x)


def _mesh():
    return plsc.VectorSubcoreMesh(core_axis_name="c", subcore_axis_name="s")


NPAD = 10240


def _deg_reduce_write(spm, sid, core_row, deg_hbm, red_v, tmp_v):
    base = sid * 640
    def zrow(t, _):
        red_v[pl.ds(t * 16, 16)] = jnp.full((16,), 2.0, jnp.float32)
        return 0
    lax.fori_loop(0, 40, zrow, 0)
    def srow(j, _):
        pltpu.sync_copy(spm.at[j, pl.ds(base, 640)], tmp_v)
        def add(t, _):
            red_v[pl.ds(t * 16, 16)] = (red_v[pl.ds(t * 16, 16)]
                                        + tmp_v[pl.ds(t * 16, 16)])
            return 0
        lax.fori_loop(0, 40, add, 0)
        return 0
    lax.fori_loop(0, NS, srow, 0)
    pltpu.sync_copy(red_v, deg_hbm.at[core_row, pl.ds(base, 640)])


def _sc_deg(dst_d3, w_d3, dst_p3, w_p3, g_d, g_p):
    def body(dst_d, w_d, dst_p, w_p, deg_hbm,
             dstv_d, wv_d, dstv_p, wv_p, deg_v, red_v, tmp_v, spm):
        cid = lax.axis_index("c")
        sid = lax.axis_index("s")
        def z(i, _):
            deg_v[pl.ds(i * 16, 16)] = jnp.zeros((16,), jnp.float32)
            return 0
        lax.fori_loop(0, NPAD // 16, z, 0)

        lanes = lax.iota(jnp.int32, 16)

        def accumulate(dstv, wv, n_grp):
            def grp(g, _):
                def b16(b, _):
                    dst16 = dstv[g, pl.ds(b * 16, 16)]
                    w16 = wv[g, pl.ds(b * 16, 16)]
                    for t in range(16):
                        d = dst16[t]
                        base = d & ~15
                        off = d & 15
                        sl = pl.ds(base, 16)
                        deg_v[sl] = deg_v[sl] + jnp.where(
                            lanes == off, w16[t], 0.0)
                    return 0
                return lax.fori_loop(0, 8, b16, 0)
            lax.fori_loop(0, n_grp, grp, 0)

        @pl.when(cid == 0)
        def _():
            pltpu.sync_copy(dst_d.at[sid], dstv_d)
            pltpu.sync_copy(w_d.at[sid], wv_d)
            accumulate(dstv_d, wv_d, g_d)

        @pl.when(cid == 1)
        def _():
            pltpu.sync_copy(dst_p.at[sid], dstv_p)
            pltpu.sync_copy(w_p.at[sid], wv_p)
            accumulate(dstv_p, wv_p, g_p)

        pltpu.sync_copy(deg_v, spm.at[sid])
        plsc.subcore_barrier()
        _deg_reduce_write(spm, sid, cid, deg_hbm, red_v, tmp_v)

    return pl.kernel(
        body,
        out_type=jax.ShapeDtypeStruct((NCORE, NPAD), jnp.float32),
        mesh=_mesh(),
        compiler_params=pltpu.CompilerParams(needs_layout_passes=False),
        scratch_types=[
            pltpu.VMEM((g_d, 128), jnp.int32),
            pltpu.VMEM((g_d, 128), jnp.float32),
            pltpu.VMEM((g_p, 128), jnp.int32),
            pltpu.VMEM((g_p, 128), jnp.float32),
            pltpu.VMEM((NPAD,), jnp.float32),
            pltpu.VMEM((640,), jnp.float32),
            pltpu.VMEM((640,), jnp.float32),
            pltpu.VMEM_SHARED((NS, NPAD), jnp.float32),
        ],
    )(dst_d3, w_d3, dst_p3, w_p3)


def _sc_msg(hs_flat, src3, dst3, w3, zeros_h, n_grp):
    def body(hs, src_h, dst_h, w_h, zeros_hbm, acc_hbm,
             dst_v, w_v, gidx2, rows0, rows1, acc_sp,
             gs0, gs1, ss0, ss1):
        rows = (rows0, rows1)
        gsem = (gs0, gs1)
        ssem = (ss0, ss1)
        cid = lax.axis_index("c")
        sid = lax.axis_index("s")
        pltpu.sync_copy(dst_h.at[sid], dst_v)
        pltpu.sync_copy(w_h.at[sid], w_v)

        def scale(rv, g):
            def b16(b, _):
                w16 = w_v[g, pl.ds(b * 16, 16)]
                for t in range(16):
                    s = w16[t]
                    e = b * 16 + t
                    for t2 in range(8):
                        sl = pl.ds(t2 * 16, 16)
                        rv[e, sl] = rv[e, sl] * s
                return 0
            lax.fori_loop(0, 8, b16, 0)

        base = sid * 624

        def chunk(j, _):
            k = 2 * j + cid
            hoff = k * N_NODES

            @pl.when(sid < NS - 1)
            def _():
                pltpu.sync_copy(zeros_hbm.at[pl.ds(0, 624)],
                                acc_sp.at[pl.ds(base, 624)])

            @pl.when(sid == NS - 1)
            def _():
                pltpu.sync_copy(zeros_hbm, acc_sp.at[pl.ds(base, 640)])

            pltpu.sync_copy(src_h.at[sid], gidx2)
            def gi(g, _):
                for t in range(8):
                    sl = pl.ds(t * 16, 16)
                    gidx2[g, sl] = gidx2[g, sl] + hoff
                return 0
            lax.fori_loop(0, n_grp, gi, 0)
            plsc.subcore_barrier()

            pltpu.async_copy(hs.at[gidx2.at[0]], rows[0], gsem[0])

            def round2(r, _):
                for b in range(2):
                    g = 2 * r + b
                    nb = 1 - b
                    pltpu.make_async_copy(hs.at[gidx2.at[g]], rows[b],
                                          gsem[b]).wait()
                    scale(rows[b], g)
                    pltpu.async_copy(rows[b], acc_sp.at[dst_v.at[g]],
                                     ssem[b], add=True)

                    @pl.when(g + 1 < n_grp)
                    def _():
                        @pl.when(g >= 1)
                        def _():
                            pltpu.make_async_copy(rows[nb],
                                                  acc_sp.at[dst_v.at[g - 1]],
                                                  ssem[nb]).wait()
                        pltpu.async_copy(hs.at[gidx2.at[g + 1]], rows[nb],
                                         gsem[nb])
                return 0
            lax.fori_loop(0, n_grp // 2, round2, 0)
            pltpu.make_async_copy(rows[0], acc_sp.at[dst_v.at[n_grp - 2]],
                                  ssem[0]).wait()
            pltpu.make_async_copy(rows[1], acc_sp.at[dst_v.at[n_grp - 1]],
                                  ssem[1]).wait()
            plsc.subcore_barrier()

            @pl.when(sid < NS - 1)
            def _():
                pltpu.sync_copy(acc_sp.at[pl.ds(base, 624)],
                                acc_hbm.at[pl.ds(hoff + base, 624)])

            @pl.when(sid == NS - 1)
            def _():
                pltpu.sync_copy(acc_sp.at[pl.ds(base, 640)],
                                acc_hbm.at[pl.ds(hoff + base, 640)])

            plsc.subcore_barrier()
            return 0
        lax.fori_loop(0, NCHUNK // NCORE, chunk, 0)

    return pl.kernel(
        body,
        out_type=jax.ShapeDtypeStruct((NCHUNK * N_NODES, CW), jnp.float32),
        mesh=_mesh(),
        compiler_params=pltpu.CompilerParams(needs_layout_passes=False),
        scratch_types=[
            pltpu.VMEM((n_grp, 128), jnp.int32),
            pltpu.VMEM((n_grp, 128), jnp.float32),
            pltpu.VMEM((n_grp, 128), jnp.int32),
            pltpu.VMEM((128, CW), jnp.float32),
            pltpu.VMEM((128, CW), jnp.float32),
            pltpu.VMEM_SHARED((NPAD, CW), jnp.float32),
        ] + [pltpu.SemaphoreType.DMA] * 4,
    )(hs_flat, src3, dst3, w3, zeros_h)


def _sc_gather(hs_d, acc_d, hs_p, acc_p, dinv2, didx2, pidx2, bd2, bp2):
    NIT = 2 * NCHUNK

    def per_core(hs, acc, idx_h, b_h, out_hbm, dinv_row,
                 dinv_v, idx_v, gidx_v, dr_v, b_v,
                 rh, ra, gh, ga, ws, sid):
        pltpu.sync_copy(dinv_row, dinv_v)
        pltpu.sync_copy(idx_h.at[sid], idx_v)
        pltpu.sync_copy(b_h, b_v)
        for g in range(2):
            for t in range(8):
                sl = pl.ds(t * 16, 16)
                dr_v[g, sl] = plsc.load_gather(dinv_v, [idx_v[g, sl]])
        for k in range(NCHUNK):
            for g in range(2):
                for t in range(8):
                    sl = pl.ds(t * 16, 16)
                    gidx_v[2 * k + g, sl] = idx_v[g, sl] + k * N_NODES

        def gather_pair(it, s):
            pltpu.async_copy(hs.at[gidx_v.at[it]], rh[s], gh[s])
            pltpu.async_copy(acc.at[gidx_v.at[it]], ra[s], ga[s])

        def out_slice(it):
            k = it // 2
            g = it % 2
            return out_hbm.at[pl.ds(sid * 256 + g * 128, 128),
                              pl.ds(k * CW, CW)]

        gather_pair(0, 0)

        def rnd(r, _):
            for s in range(2):
                it = 2 * r + s
                ns = 1 - s
                pltpu.make_async_copy(hs.at[gidx_v.at[it]], rh[s],
                                      gh[s]).wait()
                pltpu.make_async_copy(acc.at[gidx_v.at[it]], ra[s],
                                      ga[s]).wait()

                @pl.when(it + 1 < NIT)
                def _():
                    @pl.when(it >= 1)
                    def _():
                        pltpu.make_async_copy(ra[ns], out_slice(it - 1),
                                              ws[ns]).wait()
                    gather_pair(it + 1, ns)

                g = it % 2
                def b16(b, _):
                    dr16 = dr_v[g, pl.ds(b * 16, 16)]
                    for t in range(16):
                        sc = dr16[t]
                        e = b * 16 + t
                        for t2 in range(8):
                            sl = pl.ds(t2 * 16, 16)
                            x = (2.0 * rh[s][e, sl] + ra[s][e, sl]) * sc
                            x = x + b_v[it // 2, sl]
                            ra[s][e, sl] = jnp.where(x >= 0, x, 0.01 * x)
                    return 0
                lax.fori_loop(0, 8, b16, 0)
                pltpu.async_copy(ra[s], out_slice(it), ws[s])
            return 0
        lax.fori_loop(0, NIT // 2, rnd, 0)
        pltpu.make_async_copy(ra[0], out_slice(NIT - 2), ws[0]).wait()
        pltpu.make_async_copy(ra[1], out_slice(NIT - 1), ws[1]).wait()

    def body(hs_d_r, acc_d_r, hs_p_r, acc_p_r, dinv_r, didx_r, pidx_r,
             bd_r, bp_r, ec_hbm, go_hbm,
             dinv_v, idx_v, gidx_v, dr_v, b_v,
             rh0, rh1, ra0, ra1, gh0, gh1, ga0, ga1, ws0, ws1):
        cid = lax.axis_index("c")
        sid = lax.axis_index("s")
        rh = (rh0, rh1)
        ra = (ra0, ra1)
        gh = (gh0, gh1)
        ga = (ga0, ga1)
        ws = (ws0, ws1)

        @pl.when(cid == 0)
        def _():
            per_core(hs_d_r, acc_d_r, didx_r, bd_r, ec_hbm, dinv_r.at[0],
                     dinv_v, idx_v, gidx_v, dr_v, b_v, rh, ra, gh, ga, ws,
                     sid)

        @pl.when(cid == 1)
        def _():
            per_core(hs_p_r, acc_p_r, pidx_r, bp_r, go_hbm, dinv_r.at[1],
                     dinv_v, idx_v, gidx_v, dr_v, b_v, rh, ra, gh, ga, ws,
                     sid)

    return pl.kernel(
        body,
        out_type=[jax.ShapeDtypeStruct((B, D), jnp.float32),
                  jax.ShapeDtypeStruct((B, D), jnp.float32)],
        mesh=_mesh(),
        compiler_params=pltpu.CompilerParams(needs_layout_passes=False),
        scratch_types=[
            pltpu.VMEM((N_NODES,), jnp.float32),
            pltpu.VMEM((2, 128), jnp.int32),
            pltpu.VMEM((2 * NCHUNK, 128), jnp.int32),
            pltpu.VMEM((2, 128), jnp.float32),
            pltpu.VMEM((NCHUNK, CW), jnp.float32),
            pltpu.VMEM((128, CW), jnp.float32),
            pltpu.VMEM((128, CW), jnp.float32),
            pltpu.VMEM((128, CW), jnp.float32),
            pltpu.VMEM((128, CW), jnp.float32),
        ] + [pltpu.SemaphoreType.DMA] * 6,
    )(hs_d, acc_d, hs_p, acc_p, dinv2, didx2, pidx2, bd2, bp2)


def _dinv_body(deg_ref, o_ref):
    d = deg_ref[...]
    o_ref[...] = jnp.where(d > 0, lax.rsqrt(d), 0.0)


def _tc_dinv(deg2):
    return pl.pallas_call(
        _dinv_body,
        out_shape=jax.ShapeDtypeStruct((NCORE, NPAD), jnp.float32),
    )(deg2)


def _mmc_body(x_ref, w_ref, dinv_ref, o_ref):
    x = x_ref[...]
    dv = dinv_ref[...]
    for j in range(NCHUNK):
        o_ref[j] = dv * jnp.dot(x, w_ref[:, j * CW:(j + 1) * CW],
                                preferred_element_type=jnp.float32)


def _tc_matmul_chunked(x, w, dinv_col, bm):
    m, k = x.shape
    grid = (m // bm,)
    return pl.pallas_call(
        _mmc_body,
        grid=grid,
        in_specs=[
            pl.BlockSpec((bm, k), lambda i: (i, 0)),
            pl.BlockSpec((k, D), lambda i: (0, 0)),
            pl.BlockSpec((bm, 1), lambda i: (i, 0)),
        ],
        out_specs=pl.BlockSpec((NCHUNK, bm, CW), lambda i: (0, i, 0)),
        out_shape=jax.ShapeDtypeStruct((NCHUNK, m, CW), jnp.float32),
    )(x, w, dinv_col)


def _mlp_body(dv_ref, pe_ref, ec_ref, go_ref,
              w1a_ref, w1b_ref, w1c_ref, w1d_ref, b1_ref, g1_ref, be1_ref,
              w2_ref, b2_ref, g2_ref, be2_ref,
              w3_ref, b3_ref, g3_ref, be3_ref,
              w4_ref, b4_ref,
              y_ref, feat_ref):
    h = jnp.dot(dv_ref[...], w1a_ref[...], preferred_element_type=jnp.float32)
    h += jnp.dot(pe_ref[...], w1b_ref[...], preferred_element_type=jnp.float32)
    h += jnp.dot(ec_ref[...], w1c_ref[...], preferred_element_type=jnp.float32)
    h += jnp.dot(go_ref[...], w1d_ref[...], preferred_element_type=jnp.float32)
    h = h + b1_ref[...]
    h = _leaky(h * _BN_SCALE * g1_ref[...] + be1_ref[...])

    f = jnp.dot(h, w2_ref[...], preferred_element_type=jnp.float32) + b2_ref[...]
    f = _leaky(f * _BN_SCALE * g2_ref[...] + be2_ref[...])
    feat_ref[...] = f

    o = jnp.dot(f, w3_ref[...], preferred_element_type=jnp.float32) + b3_ref[...]
    o = jnp.where(o >= 0, o, o * _RRELU_SLOPE)
    o = o * _BN_SCALE * g3_ref[...] + be3_ref[...]

    y_ref[...] = jnp.dot(o, w4_ref[...], preferred_element_type=jnp.float32) \
        + b4_ref[...]


def _pallas_mlp(dv, pe, ec, go, W1, b1, g1, be1, W2, b2, g2, be2,
                W3, b3, g3, be3, W4, b4):
    bm = 512
    grid = (B // bm,)
    w1a = W1[:300]
    w1b = W1[300:1324]
    w1c = W1[1324:2348]
    w1d = W1[2348:]
    row = lambda v: v.reshape(1, -1)

    def full(a):
        return pl.BlockSpec(a.shape, lambda i: (0,) * a.ndim)

    args = (dv, pe, ec, go, w1a, w1b, w1c, w1d, row(b1), row(g1), row(be1),
            W2, row(b2), row(g2), row(be2), W3, row(b3), row(g3), row(be3),
            W4, row(b4))
    in_specs = [
        pl.BlockSpec((bm, 300), lambda i: (i, 0)),
        pl.BlockSpec((bm, 1024), lambda i: (i, 0)),
        pl.BlockSpec((bm, 1024), lambda i: (i, 0)),
        pl.BlockSpec((bm, 1024), lambda i: (i, 0)),
    ] + [full(a) for a in args[4:]]
    return pl.pallas_call(
        _mlp_body,
        grid=grid,
        in_specs=in_specs,
        out_specs=[
            pl.BlockSpec((bm, 1), lambda i: (i, 0)),
            pl.BlockSpec((bm, 512), lambda i: (i, 0)),
        ],
        out_shape=[
            jax.ShapeDtypeStruct((B, 1), jnp.float32),
            jax.ShapeDtypeStruct((B, 512), jnp.float32),
        ],
    )(*args)


def _pad_edges(edge_index, edge_weight, n_grp):
    e = edge_index.shape[1]
    cap = NS * n_grp * 128
    src = jnp.zeros((cap,), jnp.int32).at[:e].set(
        edge_index[0].astype(jnp.int32))
    dst = (10000 + jnp.arange(cap, dtype=jnp.int32) % 240).at[:e].set(
        edge_index[1].astype(jnp.int32))
    w = jnp.zeros((cap,), jnp.float32).at[:e].set(edge_weight)
    return (src.reshape(NS, n_grp, 128), dst.reshape(NS, n_grp, 128),
            w.reshape(NS, n_grp, 128))


def kernel(d_index, p_index, d_vecs, p_embeddings, y, d_ecfps, d_edge_index,
           d_edge_weight, p_gos, p_edge_index, p_edge_weight, Wd, bd, Wp, bp,
           W1, b1, g1, be1, W2, b2, g2, be2, W3, b3, g3, be3, W4, b4):
    g_d = -(-d_edge_index.shape[1] // (NS * 128))
    g_p = -(-p_edge_index.shape[1] // (NS * 128))
    src_d3, dst_d3, w_d3 = _pad_edges(d_edge_index, d_edge_weight, g_d)
    src_p3, dst_p3, w_p3 = _pad_edges(p_edge_index, p_edge_weight, g_p)

    deg2 = _sc_deg(dst_d3, w_d3, dst_p3, w_p3, g_d, g_p)
    dinv2 = _tc_dinv(deg2)[:, :N_NODES]

    hs_d = _tc_matmul_chunked(d_ecfps, Wd, dinv2[0].reshape(-1, 1), 1000)
    hs_p = _tc_matmul_chunked(p_gos, Wp, dinv2[1].reshape(-1, 1), 1000)
    hs_d_flat = hs_d.reshape(NCHUNK * N_NODES, CW)
    hs_p_flat = hs_p.reshape(NCHUNK * N_NODES, CW)

    zeros_h = jnp.zeros((640, CW), jnp.float32)
    acc_d = _sc_msg(hs_d_flat, src_d3, dst_d3, w_d3, zeros_h, g_d)
    acc_p = _sc_msg(hs_p_flat, src_p3, dst_p3, w_p3, zeros_h, g_p)

    didx2 = d_index.astype(jnp.int32).reshape(NS, 2, 128)
    pidx2 = p_index.astype(jnp.int32).reshape(NS, 2, 128)
    ec, go = _sc_gather(hs_d_flat, acc_d, hs_p_flat, acc_p, dinv2,
                        didx2, pidx2,
                        bd.reshape(NCHUNK, CW), bp.reshape(NCHUNK, CW))

    y_out, feature = _pallas_mlp(d_vecs, p_embeddings, ec, go,
                                 W1, b1, g1, be1, W2, b2, g2, be2,
                                 W3, b3, g3, be3, W4, b4)
    return (y_out, feature)
